# baseline (speedup 1.0000x reference)
import functools

import jax
import jax.numpy as jnp
from jax import lax
from jax.experimental import pallas as pl
from jax.experimental.pallas import tpu as pltpu
from jax.experimental.pallas import tpu_sc as plsc

N = 100000
NBX = 256
NBY = 256
BSX = 1.0 / NBX
BSY = 1.0 / NBY
STRETCH = 1.4142135
MINSX = BSX * STRETCH
MINSY = BSY * STRETCH
INV_CAP = 1.0 / 100.0

NW = 32
PER_W = 3136
NBATCH = PER_W // 16
LAST_START = N - PER_W
LAST_SKIP = (NW - 1) * PER_W - LAST_START
LAST_B0 = LAST_SKIP // 16

ROWS = 264
SROW = 264
HROWS = 384
ACCW = HROWS * SROW
ZEROW = ROWS * SROW


def _build_sc_scatter():
    mesh = plsc.VectorSubcoreMesh(core_axis_name="c", subcore_axis_name="s")

    @functools.partial(
        pl.kernel,
        mesh=mesh,
        compiler_params=pltpu.CompilerParams(needs_layout_passes=False),
        out_type=jax.ShapeDtypeStruct((NW, ACCW), jnp.float32),
        scratch_types=[
            pltpu.VMEM((2 * PER_W,), jnp.float32),
            pltpu.VMEM((2 * PER_W,), jnp.float32),
            pltpu.VMEM((PER_W,), jnp.float32),
            pltpu.VMEM((ACCW,), jnp.float32),
            pltpu.VMEM((272,), jnp.float32),
            pltpu.VMEM((272,), jnp.int32),
            pltpu.SemaphoreType.DMA,
        ],
    )
    def sc_fn(szf, posf, wtf, outp, szbuf, posbuf, wtbuf, acc, vbuf, ibuf, sem):
        wid = lax.axis_index("s") * 2 + lax.axis_index("c")
        is_last = wid == NW - 1
        row0 = jnp.where(is_last, LAST_START, wid * PER_W)
        b0 = jnp.where(is_last, LAST_B0, 0)
        cps = pltpu.async_copy(szf.at[pl.ds(2 * row0, 2 * PER_W)], szbuf, sem)
        cpp = pltpu.async_copy(posf.at[pl.ds(2 * row0, 2 * PER_W)], posbuf, sem)
        cpw = pltpu.async_copy(wtf.at[pl.ds(row0, PER_W)], wtbuf, sem)

        zero16 = jnp.zeros((16,), jnp.float32)

        def zblock(r, carry):
            for k in range(16):
                acc[pl.ds(r * 256 + k * 16, 16)] = zero16
            return carry

        lax.fori_loop(0, ZEROW // 256, zblock, 0, unroll=False)
        for k in range(ZEROW // 256 * 16, ZEROW // 16):
            acc[pl.ds(k * 16, 16)] = zero16
        cps.wait()
        cpp.wait()
        cpw.wait()

        iota = jnp.arange(16, dtype=jnp.int32)
        tidx = [iota * 17 + q for q in range(16)]
        even = iota * 2

        def batch(t, carry):
            o = t * 32
            x = plsc.load_gather(posbuf, [even + o])
            y = plsc.load_gather(posbuf, [even + (o + 1)])
            w = plsc.load_gather(szbuf, [even + o])
            h = plsc.load_gather(szbuf, [even + (o + 1)])
            wt = wtbuf[pl.ds(t * 16, 16)]
            sx = jnp.maximum(w, MINSX)
            sy = jnp.maximum(h, MINSY)
            dens = wt * INV_CAP / (sx * sy)
            ux = x * 256.0 - sx * 128.0
            vx = ux + sx * 256.0
            i0x = (ux + 8.0).astype(jnp.int32) - 8
            fx = ux - i0x.astype(jnp.float32)
            i1x = (vx + 8.0).astype(jnp.int32) - 8
            gx = vx - i1x.astype(jnp.float32)
            cx = (i1x - i0x) == 1
            xm = [
                (i0x + 4) * SROW,
                (i0x + 5) * SROW,
                jnp.where(cx, i0x + 3, i1x + 4) * SROW,
                (i1x + 5) * SROW,
            ]
            dxd = [
                (1.0 - fx) * dens,
                jnp.where(cx, fx - 1.0 + gx, fx) * dens,
                jnp.where(cx, 0.0, gx - 1.0) * dens,
                -gx * dens,
            ]
            uy = y * 256.0 - sy * 128.0
            vy = uy + sy * 256.0
            i0y = (uy + 8.0).astype(jnp.int32) - 8
            fy = uy - i0y.astype(jnp.float32)
            i1y = (vy + 8.0).astype(jnp.int32) - 8
            gy = vy - i1y.astype(jnp.float32)
            cy = (i1y - i0y) == 1
            yp = [
                i0y + 4,
                i0y + 5,
                jnp.where(cy, i0y + 3, i1y + 4),
                i1y + 5,
            ]
            dy = [
                1.0 - fy,
                jnp.where(cy, fy - 1.0 + gy, fy),
                jnp.where(cy, 0.0, gy - 1.0),
                -gy,
            ]
            for q in range(16):
                a, b = q >> 2, q & 3
                plsc.store_scatter(ibuf, [tidx[q]], xm[a] + yp[b])
                plsc.store_scatter(vbuf, [tidx[q]], dxd[a] * dy[b])

            @plsc.parallel_loop(0, 16, 1, unroll=16)
            def drain(j):
                off = j * 17
                iv = ibuf[pl.ds(off, 16)]
                vv = vbuf[pl.ds(off, 16)]
                plsc.addupdate_scatter(acc, [iv], vv)

            return carry

        lax.fori_loop(b0, NBATCH, batch, 0, unroll=False)
        pltpu.sync_copy(acc, outp.at[wid])

    return sc_fn


def _tc_sum_body(parts_ref, out_ref):
    out_ref[...] = jnp.sum(parts_ref[...], axis=0)


_tc_sum = pl.pallas_call(
    _tc_sum_body,
    out_shape=jax.ShapeDtypeStruct((ACCW,), jnp.float32),
)


def _tc_reduce_body(s_ref, out_ref):
    s = s_ref[...]
    rmask = lax.broadcasted_iota(jnp.int32, (HROWS, SROW), 0) < ROWS
    s = jnp.where(rmask, s, 0.0)
    c_in = lax.broadcasted_iota(jnp.int32, (NBX, HROWS), 1)
    c_out = lax.broadcasted_iota(jnp.int32, (NBX, HROWS), 0)
    amat = (c_in <= c_out + 4).astype(jnp.float32)
    d_in = lax.broadcasted_iota(jnp.int32, (SROW, NBY), 0)
    d_out = lax.broadcasted_iota(jnp.int32, (SROW, NBY), 1)
    bmat = (d_in <= d_out + 4).astype(jnp.float32)
    t = jax.lax.dot(s, bmat, precision=jax.lax.Precision.HIGHEST)
    out_ref[...] = jax.lax.dot(amat, t, precision=jax.lax.Precision.HIGHEST)


_tc_reduce = pl.pallas_call(
    _tc_reduce_body,
    out_shape=jax.ShapeDtypeStruct((NBX, NBY), jnp.float32),
)


def kernel(inst_sizes, inst_pos, inst_pin_weights):
    szf = inst_sizes.reshape(-1)
    posf = inst_pos.reshape(-1)
    parts = _build_sc_scatter()(szf, posf, inst_pin_weights)
    s = _tc_sum(parts).reshape(HROWS, SROW)
    return _tc_reduce(s)


# speedup vs baseline: 2.9846x; 2.9846x over previous
import functools

import jax
import jax.numpy as jnp
from jax import lax
from jax.experimental import pallas as pl
from jax.experimental.pallas import tpu as pltpu
from jax.experimental.pallas import tpu_sc as plsc

N = 100000
NBX = 256
NBY = 256
BSX = 1.0 / NBX
BSY = 1.0 / NBY
STRETCH = 1.4142135
MINSX = BSX * STRETCH
MINSY = BSY * STRETCH
INV_CAP = 1.0 / 100.0

NW = 32
PER_W = 3136
NBATCH = PER_W // 16
LAST_START = N - PER_W
LAST_SKIP = (NW - 1) * PER_W - LAST_START
LAST_B0 = LAST_SKIP // 16

ROWS = 264
SROW = 264
HROWS = 384
ACCW = HROWS * SROW
ZEROW = ROWS * SROW


def _build_sc_scatter():
    mesh = plsc.VectorSubcoreMesh(core_axis_name="c", subcore_axis_name="s")

    @functools.partial(
        pl.kernel,
        mesh=mesh,
        compiler_params=pltpu.CompilerParams(needs_layout_passes=False),
        out_type=jax.ShapeDtypeStruct((NW, ACCW), jnp.float32),
        scratch_types=[
            pltpu.VMEM((5 * PER_W,), jnp.float32),
            pltpu.VMEM((ACCW,), jnp.float32),
            pltpu.VMEM((272,), jnp.float32),
            pltpu.VMEM((272,), jnp.int32),
            pltpu.SemaphoreType.DMA,
        ],
    )
    def sc_fn(xf, yf, wf, hf, wtf, outp, inbuf, acc, vbuf, ibuf, sem):
        wid = lax.axis_index("s") * 2 + lax.axis_index("c")
        is_last = wid == NW - 1
        row0 = jnp.where(is_last, LAST_START, wid * PER_W)
        b0 = jnp.where(is_last, LAST_B0, 0)
        cps = [
            pltpu.async_copy(
                src.at[pl.ds(row0, PER_W)],
                inbuf.at[pl.ds(k * PER_W, PER_W)],
                sem,
            )
            for k, src in enumerate((xf, yf, wf, hf, wtf))
        ]

        zero16 = jnp.zeros((16,), jnp.float32)

        def zblock(r, carry):
            for k in range(16):
                acc[pl.ds(r * 256 + k * 16, 16)] = zero16
            return carry

        lax.fori_loop(0, ZEROW // 256, zblock, 0, unroll=False)
        for k in range(ZEROW // 256 * 16, ZEROW // 16):
            acc[pl.ds(k * 16, 16)] = zero16
        for cp in cps:
            cp.wait()

        iota = jnp.arange(16, dtype=jnp.int32)
        tidx = [iota * 17 + q for q in range(16)]

        def batch(t, carry):
            o = t * 16
            x = inbuf[pl.ds(o, 16)]
            y = inbuf[pl.ds(PER_W + o, 16)]
            w = inbuf[pl.ds(2 * PER_W + o, 16)]
            h = inbuf[pl.ds(3 * PER_W + o, 16)]
            wt = inbuf[pl.ds(4 * PER_W + o, 16)]
            sx = jnp.maximum(w, MINSX)
            sy = jnp.maximum(h, MINSY)
            dens = wt * INV_CAP / (sx * sy)
            ux = x * 256.0 - sx * 128.0
            vx = ux + sx * 256.0
            i0x = (ux + 8.0).astype(jnp.int32) - 8
            fx = ux - i0x.astype(jnp.float32)
            i1x = (vx + 8.0).astype(jnp.int32) - 8
            gx = vx - i1x.astype(jnp.float32)
            cx = (i1x - i0x) == 1
            xm = [
                (i0x + 4) * SROW,
                (i0x + 5) * SROW,
                jnp.where(cx, i0x + 3, i1x + 4) * SROW,
                (i1x + 5) * SROW,
            ]
            dxd = [
                (1.0 - fx) * dens,
                jnp.where(cx, fx - 1.0 + gx, fx) * dens,
                jnp.where(cx, 0.0, gx - 1.0) * dens,
                -gx * dens,
            ]
            uy = y * 256.0 - sy * 128.0
            vy = uy + sy * 256.0
            i0y = (uy + 8.0).astype(jnp.int32) - 8
            fy = uy - i0y.astype(jnp.float32)
            i1y = (vy + 8.0).astype(jnp.int32) - 8
            gy = vy - i1y.astype(jnp.float32)
            cy = (i1y - i0y) == 1
            yp = [
                i0y + 4,
                i0y + 5,
                jnp.where(cy, i0y + 3, i1y + 4),
                i1y + 5,
            ]
            dy = [
                1.0 - fy,
                jnp.where(cy, fy - 1.0 + gy, fy),
                jnp.where(cy, 0.0, gy - 1.0),
                -gy,
            ]
            for q in range(16):
                a, b = q >> 2, q & 3
                plsc.store_scatter(ibuf, [tidx[q]], xm[a] + yp[b])
                plsc.store_scatter(vbuf, [tidx[q]], dxd[a] * dy[b])

            @plsc.parallel_loop(0, 16, 1, unroll=16)
            def drain(j):
                off = j * 17
                iv = ibuf[pl.ds(off, 16)]
                vv = vbuf[pl.ds(off, 16)]
                plsc.addupdate_scatter(acc, [iv], vv)

            return carry

        lax.fori_loop(b0, NBATCH, batch, 0, unroll=False)
        pltpu.sync_copy(acc, outp.at[wid])

    return sc_fn


def _tc_sum_body(parts_ref, out_ref):
    out_ref[...] = jnp.sum(parts_ref[...], axis=0)


_tc_sum = pl.pallas_call(
    _tc_sum_body,
    out_shape=jax.ShapeDtypeStruct((ACCW,), jnp.float32),
)


def _tc_reduce_body(s_ref, out_ref):
    s = s_ref[...]
    rmask = lax.broadcasted_iota(jnp.int32, (HROWS, SROW), 0) < ROWS
    s = jnp.where(rmask, s, 0.0)
    c_in = lax.broadcasted_iota(jnp.int32, (NBX, HROWS), 1)
    c_out = lax.broadcasted_iota(jnp.int32, (NBX, HROWS), 0)
    amat = (c_in <= c_out + 4).astype(jnp.float32)
    d_in = lax.broadcasted_iota(jnp.int32, (SROW, NBY), 0)
    d_out = lax.broadcasted_iota(jnp.int32, (SROW, NBY), 1)
    bmat = (d_in <= d_out + 4).astype(jnp.float32)
    t = jax.lax.dot(s, bmat, precision=jax.lax.Precision.HIGHEST)
    out_ref[...] = jax.lax.dot(amat, t, precision=jax.lax.Precision.HIGHEST)


_tc_reduce = pl.pallas_call(
    _tc_reduce_body,
    out_shape=jax.ShapeDtypeStruct((NBX, NBY), jnp.float32),
)


def kernel(inst_sizes, inst_pos, inst_pin_weights):
    x = inst_pos[:, 0]
    y = inst_pos[:, 1]
    w = inst_sizes[:, 0]
    h = inst_sizes[:, 1]
    parts = _build_sc_scatter()(x, y, w, h, inst_pin_weights)
    s = _tc_sum(parts).reshape(HROWS, SROW)
    return _tc_reduce(s)
